# baseline (speedup 1.0000x reference)
import functools

import jax
import jax.numpy as jnp
from jax import lax
from jax.experimental import pallas as pl
from jax.experimental.pallas import tpu as pltpu
from jax.experimental.pallas import tpu_sc as plsc

N = 10000
E = 320000
D = 128
NPAD = 10240
BN = 512
NBLK = NPAD // BN
BE = 8000
EBLK = E // BE
NTILES = 32
CPT = D // NTILES
EPT = E // NTILES
CH = 2000
CB = 200


def _k1_body(h_ref, Wp_ref, bp_ref, Ws_ref, hp1T_ref, S1_ref):
    hblk = h_ref[...]
    hp = lax.dot_general(Wp_ref[...], hblk, (((1,), (1,)), ((), ())),
                         preferred_element_type=jnp.float32)
    hp1T_ref[...] = jnp.maximum(hp + bp_ref[...][:, None], 0.0)
    S1_ref[...] = lax.dot_general(hblk, Ws_ref[...], (((1,), (1,)), ((), ())),
                                  preferred_element_type=jnp.float32)


def _k1(h_pad, enc_Wp, enc_bp, enc_Ws):
    return pl.pallas_call(
        _k1_body,
        grid=(NBLK,),
        in_specs=[pl.BlockSpec((BN, D), lambda j: (j, 0)),
                  pl.BlockSpec((D, D), lambda j: (0, 0)),
                  pl.BlockSpec((D,), lambda j: (0,)),
                  pl.BlockSpec((D, D), lambda j: (0, 0))],
        out_specs=[pl.BlockSpec((D, BN), lambda j: (0, j)),
                   pl.BlockSpec((BN, D), lambda j: (j, 0))],
        out_shape=[jax.ShapeDtypeStruct((D, NPAD), jnp.float32),
                   jax.ShapeDtypeStruct((NPAD, D), jnp.float32)],
    )(h_pad, enc_Wp, enc_bp, enc_Ws)


def _k2_body(S1_ref, n1T_ref, Wn_ref, b_ref, dWp_ref, dbp_ref, dWs_ref,
             npW_ref, npb_ref, npg_ref, npbeta_ref,
             W1hs_ref, W1cs_ref, W1hd_ref, W1cd_ref,
             h1_ref, hp2T_ref, S2_ref, npred_ref, A_ref, B_ref):
    t = jnp.dot(Wn_ref[...], n1T_ref[...], preferred_element_type=jnp.float32)
    h1T = jnp.maximum(t + S1_ref[...].T + b_ref[...][:, None], 0.0)
    h1 = h1T.T
    h1_ref[...] = h1
    hp2T_ref[...] = jnp.maximum(
        jnp.dot(dWp_ref[...], h1T, preferred_element_type=jnp.float32)
        + dbp_ref[...][:, None], 0.0)
    S2_ref[...] = jnp.dot(dWs_ref[...], h1T,
                          preferred_element_type=jnp.float32).T
    z = lax.dot_general(h1, npW_ref[...], (((1,), (1,)), ((), ())),
                        preferred_element_type=jnp.float32) + npb_ref[...]
    mu = jnp.mean(z, axis=1, keepdims=True)
    var = jnp.mean((z - mu) ** 2, axis=1, keepdims=True)
    npred = (z - mu) * lax.rsqrt(var + 1e-5) * npg_ref[...] + npbeta_ref[...]
    npred_ref[...] = npred
    m = jnp.max(npred, axis=1, keepdims=True)
    ez = jnp.exp(npred - m)
    cls = ez / jnp.sum(ez, axis=1, keepdims=True)
    A_ref[...] = (
        lax.dot_general(h1, W1hs_ref[...], (((1,), (1,)), ((), ())),
                        preferred_element_type=jnp.float32)
        + lax.dot_general(cls, W1cs_ref[...], (((1,), (1,)), ((), ())),
                          preferred_element_type=jnp.float32))
    B_ref[...] = (
        lax.dot_general(h1, W1hd_ref[...], (((1,), (1,)), ((), ())),
                        preferred_element_type=jnp.float32)
        + lax.dot_general(cls, W1cd_ref[...], (((1,), (1,)), ((), ())),
                          preferred_element_type=jnp.float32))


def _k2(S1, n1T, enc_Wn, enc_b, dec_Wp, dec_bp, dec_Ws,
        np_W, npb, npg, npbeta, W1hs, W1cs, W1hd, W1cd):
    full = lambda shape: pl.BlockSpec(shape, lambda j, _s=shape: tuple(0 for _ in _s))
    return pl.pallas_call(
        _k2_body,
        grid=(NBLK,),
        in_specs=[pl.BlockSpec((BN, D), lambda j: (j, 0)),
                  pl.BlockSpec((D, BN), lambda j: (0, j)),
                  full((D, D)), full((D,)), full((D, D)), full((D,)),
                  full((D, D)), full((6, D)), full((1, 6)), full((1, 6)),
                  full((1, 6)), full((D, D)), full((D, 6)), full((D, D)),
                  full((D, 6))],
        out_specs=[pl.BlockSpec((BN, D), lambda j: (j, 0)),
                   pl.BlockSpec((D, BN), lambda j: (0, j)),
                   pl.BlockSpec((BN, D), lambda j: (j, 0)),
                   pl.BlockSpec((BN, 6), lambda j: (j, 0)),
                   pl.BlockSpec((BN, D), lambda j: (j, 0)),
                   pl.BlockSpec((BN, D), lambda j: (j, 0))],
        out_shape=[jax.ShapeDtypeStruct((NPAD, D), jnp.float32),
                   jax.ShapeDtypeStruct((D, NPAD), jnp.float32),
                   jax.ShapeDtypeStruct((NPAD, D), jnp.float32),
                   jax.ShapeDtypeStruct((NPAD, 6), jnp.float32),
                   jax.ShapeDtypeStruct((NPAD, D), jnp.float32),
                   jax.ShapeDtypeStruct((NPAD, D), jnp.float32)],
    )(S1, n1T, enc_Wn, enc_b, dec_Wp, dec_bp, dec_Ws,
      np_W, npb, npg, npbeta, W1hs, W1cs, W1hd, W1cd)


def _k3_body(pre_ref, ea_ref, W1ea_ref, b1_ref, g_ref, beta_ref,
             W2_ref, b2_ref, out_ref):
    x = (pre_ref[...]
         + lax.dot_general(ea_ref[...], W1ea_ref[...], (((1,), (1,)), ((), ())),
                           preferred_element_type=jnp.float32)
         + b1_ref[...])
    mu = jnp.mean(x, axis=1, keepdims=True)
    var = jnp.mean((x - mu) ** 2, axis=1, keepdims=True)
    xn = jnp.maximum((x - mu) * lax.rsqrt(var + 1e-5) * g_ref[...]
                     + beta_ref[...], 0.0)
    out_ref[...] = lax.dot_general(xn, W2_ref[...], (((1,), (1,)), ((), ())),
                                   preferred_element_type=jnp.float32) + b2_ref[...]


def _k3(pre, edge_attr, W1ea, b1, g, beta, W2, b2):
    full = lambda shape: pl.BlockSpec(shape, lambda j, _s=shape: tuple(0 for _ in _s))
    return pl.pallas_call(
        _k3_body,
        grid=(EBLK,),
        in_specs=[pl.BlockSpec((BE, D), lambda j: (j, 0)),
                  pl.BlockSpec((BE, 4), lambda j: (j, 0)),
                  full((D, 4)), full((1, D)), full((1, D)), full((1, D)),
                  full((2, D)), full((1, 2))],
        out_specs=pl.BlockSpec((BE, 2), lambda j: (j, 0)),
        out_shape=jax.ShapeDtypeStruct((E, 2), jnp.float32),
    )(pre, edge_attr, W1ea, b1, g, beta, W2, b2)


def _k4_body(S2_ref, n2T_ref, Wn_ref, b_ref, out_ref):
    t = jnp.dot(Wn_ref[...], n2T_ref[...], preferred_element_type=jnp.float32)
    out_ref[...] = jnp.maximum(S2_ref[...] + t.T + b_ref[...][None, :], 0.0)


def _k4(S2, n2T, dec_Wn, dec_b):
    full = lambda shape: pl.BlockSpec(shape, lambda j, _s=shape: tuple(0 for _ in _s))
    return pl.pallas_call(
        _k4_body,
        grid=(NBLK,),
        in_specs=[pl.BlockSpec((BN, D), lambda j: (j, 0)),
                  pl.BlockSpec((D, BN), lambda j: (0, j)),
                  full((D, D)), full((D,))],
        out_specs=pl.BlockSpec((BN, D), lambda j: (j, 0)),
        out_shape=jax.ShapeDtypeStruct((NPAD, D), jnp.float32),
    )(S2, n2T, dec_Wn, dec_b)


def _segmax_sc(hpT_flat, src, dst):
    mesh = plsc.VectorSubcoreMesh(core_axis_name="c", subcore_axis_name="s")

    @functools.partial(
        pl.kernel,
        mesh=mesh,
        out_type=jax.ShapeDtypeStruct((D * NPAD,), jnp.float32),
        compiler_params=pltpu.CompilerParams(needs_layout_passes=False),
        scratch_types=[
            pltpu.VMEM((CPT * NPAD,), jnp.float32),
            pltpu.VMEM((CPT * NPAD,), jnp.float32),
            pltpu.VMEM((CH,), jnp.int32),
            pltpu.VMEM((CH,), jnp.int32),
            pltpu.VMEM((NPAD,), jnp.int32),
        ],
    )
    def k(hpT_hbm, src_hbm, dst_hbm, out_hbm, colbuf, acc, sbuf, dbuf, dscr):
        wid = lax.axis_index("s") * 2 + lax.axis_index("c")
        base = wid * (CPT * NPAD)
        pltpu.sync_copy(hpT_hbm.at[pl.ds(base, CPT * NPAD)], colbuf)

        def zb(i, carry):
            acc[pl.ds(i * 16, 16)] = jnp.zeros((16,), jnp.float32)
            return carry
        lax.fori_loop(0, CPT * NPAD // 16, zb, 0)

        iota = lax.iota(jnp.int32, 16)

        def chunk_body(kk, carry):
            pltpu.sync_copy(src_hbm.at[pl.ds(kk * CH, CH)], sbuf)
            pltpu.sync_copy(dst_hbm.at[pl.ds(kk * CH, CH)], dbuf)

            def grp(g, carry2):
                s = sbuf[pl.ds(g * 16, 16)]
                d = dbuf[pl.ds(g * 16, 16)]
                plsc.store_scatter(dscr, [d], iota)
                back = plsc.load_gather(dscr, [d])
                hasdup = jnp.any(back != iota)

                @pl.when(jnp.logical_not(hasdup))
                def _():
                    for c in range(CPT):
                        dc = d + (c * NPAD)
                        val = plsc.load_gather(colbuf, [s + (c * NPAD)])
                        cur = plsc.load_gather(acc, [dc])
                        plsc.store_scatter(acc, [dc], jnp.maximum(cur, val))

                @pl.when(hasdup)
                def _():
                    sd, perm = plsc.sort_key_val(d, iota)
                    sp = s.at[perm].get(mode="promise_in_bounds")
                    vals = [plsc.load_gather(colbuf, [sp + (c * NPAD)])
                            for c in range(CPT)]
                    for sh in (1, 2, 4, 8):
                        ish = jnp.maximum(iota - sh, 0)
                        ksh = sd.at[ish].get(mode="promise_in_bounds")
                        msh = (iota >= sh) & (ksh == sd)
                        for c in range(CPT):
                            vv = vals[c].at[ish].get(mode="promise_in_bounds")
                            vals[c] = jnp.where(msh,
                                               jnp.maximum(vals[c], vv),
                                               vals[c])
                    nxt = sd.at[jnp.minimum(iota + 1, 15)].get(
                        mode="promise_in_bounds")
                    is_last = (iota == 15) | (nxt != sd)
                    for c in range(CPT):
                        dc = sd + (c * NPAD)
                        cur = plsc.load_gather(acc, [dc])
                        plsc.store_scatter(acc, [dc],
                                           jnp.maximum(cur, vals[c]),
                                           mask=is_last)
                return carry2
            lax.fori_loop(0, CH // 16, grp, 0)
            return carry
        lax.fori_loop(0, E // CH, chunk_body, 0)
        pltpu.sync_copy(acc, out_hbm.at[pl.ds(base, CPT * NPAD)])

    return k(hpT_flat, src, dst)


def _edge_gather_sc(A, B, src, dst):
    mesh = plsc.VectorSubcoreMesh(core_axis_name="c", subcore_axis_name="s")

    @functools.partial(
        pl.kernel,
        mesh=mesh,
        out_type=jax.ShapeDtypeStruct((E, D), jnp.float32),
        compiler_params=pltpu.CompilerParams(needs_layout_passes=False),
        scratch_types=[
            pltpu.VMEM((CB,), jnp.int32),
            pltpu.VMEM((CB,), jnp.int32),
            pltpu.VMEM((CB, D), jnp.float32),
            pltpu.VMEM((CB, D), jnp.float32),
            pltpu.SemaphoreType.DMA,
            pltpu.SemaphoreType.DMA,
        ],
    )
    def k(A_hbm, B_hbm, src_hbm, dst_hbm, out_hbm,
          sidx, didx, bufA, bufB, semA, semB):
        wid = lax.axis_index("s") * 2 + lax.axis_index("c")
        base = wid * EPT

        def chunk(kk, carry):
            off = base + kk * CB
            pltpu.sync_copy(src_hbm.at[pl.ds(off, CB)], sidx)
            pltpu.sync_copy(dst_hbm.at[pl.ds(off, CB)], didx)
            cpA = pltpu.async_copy(A_hbm.at[sidx], bufA, semA)
            cpB = pltpu.async_copy(B_hbm.at[didx], bufB, semB)
            cpA.wait()
            cpB.wait()

            def addrow(r, carry2):
                for c in range(D // 16):
                    sl = pl.ds(c * 16, 16)
                    bufA[r, sl] = bufA[r, sl] + bufB[r, sl]
                return carry2
            lax.fori_loop(0, CB, addrow, 0)
            pltpu.sync_copy(bufA, out_hbm.at[pl.ds(off, CB)])
            return carry
        lax.fori_loop(0, EPT // CB, chunk, 0)

    return k(A, B, src, dst)


def kernel(h, edge_index, edge_attr,
           enc_Wp, enc_bp, enc_Ws, enc_Wn, enc_b,
           dec_Wp, dec_bp, dec_Ws, dec_Wn, dec_b,
           np_W, np_b, np_g, np_beta,
           ep_W1, ep_b1, ep_g, ep_beta, ep_W2, ep_b2):
    src = edge_index[0]
    dst = edge_index[1]
    h_pad = jnp.pad(h, ((0, NPAD - N), (0, 0)))

    hp1T, S1 = _k1(h_pad, enc_Wp, enc_bp, enc_Ws)
    n1T = _segmax_sc(hp1T.reshape(-1), src, dst).reshape(D, NPAD)

    W1hs = ep_W1[:, 0:D]
    W1cs = ep_W1[:, D:D + 6]
    W1ea = ep_W1[:, D + 6:D + 10]
    W1hd = ep_W1[:, D + 10:2 * D + 10]
    W1cd = ep_W1[:, 2 * D + 10:2 * D + 16]

    h1, hp2T, S2, npred, A, B = _k2(
        S1, n1T, enc_Wn, enc_b, dec_Wp, dec_bp, dec_Ws,
        np_W, np_b.reshape(1, 6), np_g.reshape(1, 6), np_beta.reshape(1, 6),
        W1hs, W1cs, W1hd, W1cd)

    pre = _edge_gather_sc(A, B, src, dst)
    n2T = _segmax_sc(hp2T.reshape(-1), src, dst).reshape(D, NPAD)

    edge_pred = _k3(pre, edge_attr, W1ea, ep_b1.reshape(1, D),
                    ep_g.reshape(1, D), ep_beta.reshape(1, D),
                    ep_W2, ep_b2.reshape(1, 2))
    h2 = _k4(S2, n2T, dec_Wn, dec_b)

    return (npred[:N], edge_pred, h2[:N])


# speedup vs baseline: 1.6297x; 1.6297x over previous
import functools

import jax
import jax.numpy as jnp
from jax import lax
from jax.experimental import pallas as pl
from jax.experimental.pallas import tpu as pltpu
from jax.experimental.pallas import tpu_sc as plsc

N = 10000
E = 320000
D = 128
NPAD = 10240
BN = 512
NBLK = NPAD // BN
BE = 8000
EBLK = E // BE
NTILES = 32
CPT = D // NTILES
EPT = E // NTILES
CH = 2560
CB = 200


def _k1_body(h_ref, Wp_ref, bp_ref, Ws_ref, hp1T_ref, S1_ref):
    hblk = h_ref[...]
    hp = lax.dot_general(Wp_ref[...], hblk, (((1,), (1,)), ((), ())),
                         preferred_element_type=jnp.float32)
    hp1T_ref[...] = jnp.maximum(hp + bp_ref[...][:, None], 0.0)
    S1_ref[...] = lax.dot_general(hblk, Ws_ref[...], (((1,), (1,)), ((), ())),
                                  preferred_element_type=jnp.float32)


def _k1(h_pad, enc_Wp, enc_bp, enc_Ws):
    return pl.pallas_call(
        _k1_body,
        grid=(NBLK,),
        in_specs=[pl.BlockSpec((BN, D), lambda j: (j, 0)),
                  pl.BlockSpec((D, D), lambda j: (0, 0)),
                  pl.BlockSpec((D,), lambda j: (0,)),
                  pl.BlockSpec((D, D), lambda j: (0, 0))],
        out_specs=[pl.BlockSpec((D, BN), lambda j: (0, j)),
                   pl.BlockSpec((BN, D), lambda j: (j, 0))],
        out_shape=[jax.ShapeDtypeStruct((D, NPAD), jnp.float32),
                   jax.ShapeDtypeStruct((NPAD, D), jnp.float32)],
    )(h_pad, enc_Wp, enc_bp, enc_Ws)


def _k2_body(S1_ref, n1T_ref, Wn_ref, b_ref, dWp_ref, dbp_ref, dWs_ref,
             npW_ref, npb_ref, npg_ref, npbeta_ref,
             W1hs_ref, W1cs_ref, W1hd_ref, W1cd_ref,
             h1_ref, hp2T_ref, S2_ref, npred_ref, A_ref, B_ref):
    t = jnp.dot(Wn_ref[...], n1T_ref[...], preferred_element_type=jnp.float32)
    h1T = jnp.maximum(t + S1_ref[...].T + b_ref[...][:, None], 0.0)
    h1 = h1T.T
    h1_ref[...] = h1
    hp2T_ref[...] = jnp.maximum(
        jnp.dot(dWp_ref[...], h1T, preferred_element_type=jnp.float32)
        + dbp_ref[...][:, None], 0.0)
    S2_ref[...] = jnp.dot(dWs_ref[...], h1T,
                          preferred_element_type=jnp.float32).T
    z = lax.dot_general(h1, npW_ref[...], (((1,), (1,)), ((), ())),
                        preferred_element_type=jnp.float32) + npb_ref[...]
    mu = jnp.mean(z, axis=1, keepdims=True)
    var = jnp.mean((z - mu) ** 2, axis=1, keepdims=True)
    npred = (z - mu) * lax.rsqrt(var + 1e-5) * npg_ref[...] + npbeta_ref[...]
    npred_ref[...] = npred
    m = jnp.max(npred, axis=1, keepdims=True)
    ez = jnp.exp(npred - m)
    cls = ez / jnp.sum(ez, axis=1, keepdims=True)
    A_ref[...] = (
        lax.dot_general(h1, W1hs_ref[...], (((1,), (1,)), ((), ())),
                        preferred_element_type=jnp.float32)
        + lax.dot_general(cls, W1cs_ref[...], (((1,), (1,)), ((), ())),
                          preferred_element_type=jnp.float32))
    B_ref[...] = (
        lax.dot_general(h1, W1hd_ref[...], (((1,), (1,)), ((), ())),
                        preferred_element_type=jnp.float32)
        + lax.dot_general(cls, W1cd_ref[...], (((1,), (1,)), ((), ())),
                          preferred_element_type=jnp.float32))


def _k2(S1, n1T, enc_Wn, enc_b, dec_Wp, dec_bp, dec_Ws,
        np_W, npb, npg, npbeta, W1hs, W1cs, W1hd, W1cd):
    full = lambda shape: pl.BlockSpec(shape, lambda j, _s=shape: tuple(0 for _ in _s))
    return pl.pallas_call(
        _k2_body,
        grid=(NBLK,),
        in_specs=[pl.BlockSpec((BN, D), lambda j: (j, 0)),
                  pl.BlockSpec((D, BN), lambda j: (0, j)),
                  full((D, D)), full((D,)), full((D, D)), full((D,)),
                  full((D, D)), full((6, D)), full((1, 6)), full((1, 6)),
                  full((1, 6)), full((D, D)), full((D, 6)), full((D, D)),
                  full((D, 6))],
        out_specs=[pl.BlockSpec((BN, D), lambda j: (j, 0)),
                   pl.BlockSpec((D, BN), lambda j: (0, j)),
                   pl.BlockSpec((BN, D), lambda j: (j, 0)),
                   pl.BlockSpec((BN, 6), lambda j: (j, 0)),
                   pl.BlockSpec((BN, D), lambda j: (j, 0)),
                   pl.BlockSpec((BN, D), lambda j: (j, 0))],
        out_shape=[jax.ShapeDtypeStruct((NPAD, D), jnp.float32),
                   jax.ShapeDtypeStruct((D, NPAD), jnp.float32),
                   jax.ShapeDtypeStruct((NPAD, D), jnp.float32),
                   jax.ShapeDtypeStruct((NPAD, 6), jnp.float32),
                   jax.ShapeDtypeStruct((NPAD, D), jnp.float32),
                   jax.ShapeDtypeStruct((NPAD, D), jnp.float32)],
    )(S1, n1T, enc_Wn, enc_b, dec_Wp, dec_bp, dec_Ws,
      np_W, npb, npg, npbeta, W1hs, W1cs, W1hd, W1cd)


def _k3_body(pre_ref, ea_ref, W1ea_ref, b1_ref, g_ref, beta_ref,
             W2_ref, b2_ref, out_ref):
    x = (pre_ref[...]
         + lax.dot_general(ea_ref[...], W1ea_ref[...], (((1,), (1,)), ((), ())),
                           preferred_element_type=jnp.float32)
         + b1_ref[...])
    mu = jnp.mean(x, axis=1, keepdims=True)
    var = jnp.mean((x - mu) ** 2, axis=1, keepdims=True)
    xn = jnp.maximum((x - mu) * lax.rsqrt(var + 1e-5) * g_ref[...]
                     + beta_ref[...], 0.0)
    out_ref[...] = lax.dot_general(xn, W2_ref[...], (((1,), (1,)), ((), ())),
                                   preferred_element_type=jnp.float32) + b2_ref[...]


def _k3(pre, edge_attr, W1ea, b1, g, beta, W2, b2):
    full = lambda shape: pl.BlockSpec(shape, lambda j, _s=shape: tuple(0 for _ in _s))
    return pl.pallas_call(
        _k3_body,
        grid=(EBLK,),
        in_specs=[pl.BlockSpec((BE, D), lambda j: (j, 0)),
                  pl.BlockSpec((BE, 4), lambda j: (j, 0)),
                  full((D, 4)), full((1, D)), full((1, D)), full((1, D)),
                  full((2, D)), full((1, 2))],
        out_specs=pl.BlockSpec((BE, 2), lambda j: (j, 0)),
        out_shape=jax.ShapeDtypeStruct((E, 2), jnp.float32),
    )(pre, edge_attr, W1ea, b1, g, beta, W2, b2)


def _k4_body(S2_ref, n2T_ref, Wn_ref, b_ref, out_ref):
    t = jnp.dot(Wn_ref[...], n2T_ref[...], preferred_element_type=jnp.float32)
    out_ref[...] = jnp.maximum(S2_ref[...] + t.T + b_ref[...][None, :], 0.0)


def _k4(S2, n2T, dec_Wn, dec_b):
    full = lambda shape: pl.BlockSpec(shape, lambda j, _s=shape: tuple(0 for _ in _s))
    return pl.pallas_call(
        _k4_body,
        grid=(NBLK,),
        in_specs=[pl.BlockSpec((BN, D), lambda j: (j, 0)),
                  pl.BlockSpec((D, BN), lambda j: (0, j)),
                  full((D, D)), full((D,))],
        out_specs=pl.BlockSpec((BN, D), lambda j: (j, 0)),
        out_shape=jax.ShapeDtypeStruct((NPAD, D), jnp.float32),
    )(S2, n2T, dec_Wn, dec_b)


def _segmax_sc(hpT_flat, src, dst):
    mesh = plsc.VectorSubcoreMesh(core_axis_name="c", subcore_axis_name="s")

    @functools.partial(
        pl.kernel,
        mesh=mesh,
        out_type=jax.ShapeDtypeStruct((D * NPAD,), jnp.float32),
        compiler_params=pltpu.CompilerParams(needs_layout_passes=False),
        scratch_types=[
            pltpu.VMEM((CPT * NPAD,), jnp.float32),
            pltpu.VMEM((CPT * NPAD,), jnp.float32),
            pltpu.VMEM((CH,), jnp.int32),
            pltpu.VMEM((CH,), jnp.int32),
            pltpu.VMEM((CH,), jnp.int32),
            pltpu.VMEM((CH,), jnp.int32),
            pltpu.SemaphoreType.DMA,
            pltpu.SemaphoreType.DMA,
            pltpu.SemaphoreType.DMA,
            pltpu.SemaphoreType.DMA,
        ],
    )
    def k(hpT_hbm, src_hbm, dst_hbm, out_hbm, colbuf, acc,
          sbuf0, dbuf0, sbuf1, dbuf1, semS0, semD0, semS1, semD1):
        wid = lax.axis_index("s") * 2 + lax.axis_index("c")
        base = wid * (CPT * NPAD)
        pltpu.sync_copy(hpT_hbm.at[pl.ds(base, CPT * NPAD)], colbuf)

        def zb(i, carry):
            acc[pl.ds(i * 16, 16)] = jnp.zeros((16,), jnp.float32)
            return carry
        lax.fori_loop(0, CPT * NPAD // 16, zb, 0)

        iota = lax.iota(jnp.int32, 16)

        def grp_full(g, carry2, sbuf, dbuf):
            s = sbuf[pl.ds(g * 16, 16)]
            d = dbuf[pl.ds(g * 16, 16)]
            sd, sp = plsc.sort_key_val(d, s)
            vals = [plsc.load_gather(colbuf, [sp + (c * NPAD)])
                    for c in range(CPT)]
            for sh in (1, 2, 4, 8):
                ish = jnp.maximum(iota - sh, 0)
                ksh = sd.at[ish].get(mode="promise_in_bounds")
                msh = (iota >= sh) & (ksh == sd)
                for c in range(CPT):
                    vv = vals[c].at[ish].get(mode="promise_in_bounds")
                    vals[c] = jnp.where(msh, jnp.maximum(vals[c], vv),
                                        vals[c])
            nxt = sd.at[jnp.minimum(iota + 1, 15)].get(
                mode="promise_in_bounds")
            is_last = (iota == 15) | (nxt != sd)
            for c in range(CPT):
                dc = sd + (c * NPAD)
                cur = plsc.load_gather(acc, [dc])
                plsc.store_scatter(acc, [dc], jnp.maximum(cur, vals[c]),
                                   mask=is_last)
            return carry2

        def grp_short(g, sticky, sbuf, dbuf):
            s = sbuf[pl.ds(g * 16, 16)]
            d = dbuf[pl.ds(g * 16, 16)]
            sd, sp = plsc.sort_key_val(d, s)
            vals = [plsc.load_gather(colbuf, [sp + (c * NPAD)])
                    for c in range(CPT)]
            pk = sd.at[jnp.maximum(iota - 1, 0)].get(
                mode="promise_in_bounds")
            same_prev = (sd == pk) & (iota >= 1)
            pk2 = sd.at[jnp.maximum(iota - 2, 0)].get(
                mode="promise_in_bounds")
            run3 = same_prev & (sd == pk2) & (iota >= 2)
            ish = iota - same_prev.astype(jnp.int32)
            for c in range(CPT):
                pv = vals[c].at[ish].get(mode="promise_in_bounds")
                vals[c] = jnp.maximum(vals[c], pv)
            nxt = sd.at[jnp.minimum(iota + 1, 15)].get(
                mode="promise_in_bounds")
            is_last = (iota == 15) | (nxt != sd)
            for c in range(CPT):
                dc = sd + (c * NPAD)
                cur = plsc.load_gather(acc, [dc])
                plsc.store_scatter(acc, [dc], jnp.maximum(cur, vals[c]),
                                   mask=is_last)
            return sticky | run3

        sets = ((sbuf0, dbuf0, semS0, semD0), (sbuf1, dbuf1, semS1, semD1))

        def issue(kk, st):
            sb, db, ss, sd_ = st
            pltpu.async_copy(src_hbm.at[pl.ds(kk * CH, CH)], sb, ss)
            pltpu.async_copy(dst_hbm.at[pl.ds(kk * CH, CH)], db, sd_)

        def proc(kk, st):
            sb, db, ss, sd_ = st
            pltpu.make_async_copy(src_hbm.at[pl.ds(kk * CH, CH)], sb,
                                  ss).wait()
            pltpu.make_async_copy(dst_hbm.at[pl.ds(kk * CH, CH)], db,
                                  sd_).wait()

            def g2body(g2, sticky):
                sticky = grp_short(g2 * 2, sticky, sb, db)
                return grp_short(g2 * 2 + 1, sticky, sb, db)
            sticky = lax.fori_loop(0, CH // 32, g2body,
                                   jnp.zeros((16,), jnp.bool_))

            @pl.when(jnp.any(sticky))
            def _():
                lax.fori_loop(0, CH // 16,
                              lambda g, c: grp_full(g, c, sb, db), 0)

        nch = E // CH
        issue(0, sets[0])

        def body(t, carry):
            issue(2 * t + 1, sets[1])
            proc(2 * t, sets[0])
            issue(2 * t + 2, sets[0])
            proc(2 * t + 1, sets[1])
            return carry
        lax.fori_loop(0, (nch - 1) // 2 - 1, body, 0)
        issue(nch - 2, sets[1])
        proc(nch - 3, sets[0])
        issue(nch - 1, sets[0])
        proc(nch - 2, sets[1])
        proc(nch - 1, sets[0])
        pltpu.sync_copy(acc, out_hbm.at[pl.ds(base, CPT * NPAD)])

    return k(hpT_flat, src, dst)


def _edge_gather_sc(A, B, src, dst):
    mesh = plsc.VectorSubcoreMesh(core_axis_name="c", subcore_axis_name="s")

    @functools.partial(
        pl.kernel,
        mesh=mesh,
        out_type=jax.ShapeDtypeStruct((E, D), jnp.float32),
        compiler_params=pltpu.CompilerParams(needs_layout_passes=False),
        scratch_types=[
            pltpu.VMEM((CB,), jnp.int32),
            pltpu.VMEM((CB,), jnp.int32),
            pltpu.VMEM((CB,), jnp.int32),
            pltpu.VMEM((CB,), jnp.int32),
            pltpu.VMEM((CB, D), jnp.float32),
            pltpu.VMEM((CB, D), jnp.float32),
            pltpu.VMEM((CB, D), jnp.float32),
            pltpu.VMEM((CB, D), jnp.float32),
            pltpu.SemaphoreType.DMA,
            pltpu.SemaphoreType.DMA,
            pltpu.SemaphoreType.DMA,
            pltpu.SemaphoreType.DMA,
        ],
    )
    def k(A_hbm, B_hbm, src_hbm, dst_hbm, out_hbm,
          sidx0, didx0, sidx1, didx1, bufA0, bufB0, bufA1, bufB1,
          semA0, semB0, semA1, semB1):
        wid = lax.axis_index("s") * 2 + lax.axis_index("c")
        base = wid * EPT
        sets = ((sidx0, didx0, bufA0, bufB0, semA0, semB0),
                (sidx1, didx1, bufA1, bufB1, semA1, semB1))

        def issue(kk, st):
            sidx, didx, bufA, bufB, semA, semB = st
            off = base + kk * CB
            pltpu.sync_copy(src_hbm.at[pl.ds(off, CB)], sidx)
            pltpu.sync_copy(dst_hbm.at[pl.ds(off, CB)], didx)
            pltpu.async_copy(A_hbm.at[sidx], bufA, semA)
            pltpu.async_copy(B_hbm.at[didx], bufB, semB)

        def waitproc(kk, st):
            sidx, didx, bufA, bufB, semA, semB = st
            pltpu.make_async_copy(A_hbm.at[sidx], bufA, semA).wait()
            pltpu.make_async_copy(B_hbm.at[didx], bufB, semB).wait()

            @plsc.parallel_loop(0, CB, 1, unroll=4)
            def addrow(r):
                for c in range(D // 16):
                    sl = pl.ds(c * 16, 16)
                    bufA[r, sl] = bufA[r, sl] + bufB[r, sl]
            pltpu.sync_copy(bufA, out_hbm.at[pl.ds(base + kk * CB, CB)])

        nch = EPT // CB
        issue(0, sets[0])

        def body(t, carry):
            issue(2 * t + 1, sets[1])
            waitproc(2 * t, sets[0])
            issue(2 * t + 2, sets[0])
            waitproc(2 * t + 1, sets[1])
            return carry
        lax.fori_loop(0, nch // 2 - 1, body, 0)
        issue(nch - 1, sets[1])
        waitproc(nch - 2, sets[0])
        waitproc(nch - 1, sets[1])

    return k(A, B, src, dst)


def kernel(h, edge_index, edge_attr,
           enc_Wp, enc_bp, enc_Ws, enc_Wn, enc_b,
           dec_Wp, dec_bp, dec_Ws, dec_Wn, dec_b,
           np_W, np_b, np_g, np_beta,
           ep_W1, ep_b1, ep_g, ep_beta, ep_W2, ep_b2):
    src = edge_index[0]
    dst = edge_index[1]
    h_pad = jnp.pad(h, ((0, NPAD - N), (0, 0)))

    hp1T, S1 = _k1(h_pad, enc_Wp, enc_bp, enc_Ws)
    n1T = _segmax_sc(hp1T.reshape(-1), src, dst).reshape(D, NPAD)

    W1hs = ep_W1[:, 0:D]
    W1cs = ep_W1[:, D:D + 6]
    W1ea = ep_W1[:, D + 6:D + 10]
    W1hd = ep_W1[:, D + 10:2 * D + 10]
    W1cd = ep_W1[:, 2 * D + 10:2 * D + 16]

    h1, hp2T, S2, npred, A, B = _k2(
        S1, n1T, enc_Wn, enc_b, dec_Wp, dec_bp, dec_Ws,
        np_W, np_b.reshape(1, 6), np_g.reshape(1, 6), np_beta.reshape(1, 6),
        W1hs, W1cs, W1hd, W1cd)

    pre = _edge_gather_sc(A, B, src, dst)
    n2T = _segmax_sc(hp2T.reshape(-1), src, dst).reshape(D, NPAD)

    edge_pred = _k3(pre, edge_attr, W1ea, ep_b1.reshape(1, D),
                    ep_g.reshape(1, D), ep_beta.reshape(1, D),
                    ep_W2, ep_b2.reshape(1, 2))
    h2 = _k4(S2, n2T, dec_Wn, dec_b)

    return (npred[:N], edge_pred, h2[:N])
